# baseline (speedup 1.0000x reference)
import dataclasses
import functools

import jax
import jax.numpy as jnp
import numpy as np
from jax import lax
from jax.experimental import pallas as pl
from jax.experimental.pallas import tpu as pltpu
from jax.experimental.pallas import tpu_sc as plsc

N_NODES = 10000
N_EDGES = 320000
D_FEAT = 128
HID = 64

NC = 2
NS = 16
NW = NC * NS
CHUNK = 128
NCHUNK = 2560
NCHUNK_W = NCHUNK // NW
NCHUNK_A = 100
NCHUNK_B = 60
C0_CHUNKS = NS * NCHUNK_A
PAD_ROW = 0
PAD_COL = N_NODES
N_PAD = 10240
SLICE = N_PAD // NS
NSLOT = 4

_pi32 = np.concatenate([np.arange(0, 32, 2), np.arange(1, 32, 2)])
_PI = np.concatenate([_pi32, 32 + _pi32])
_IPI = np.argsort(_PI)

_mesh = plsc.VectorSubcoreMesh(core_axis_name="c", subcore_axis_name="s")
_sc_params = pltpu.CompilerParams(use_tc_tiling_on_sc=False)
_sc_params_nl = dataclasses.replace(_sc_params, needs_layout_passes=False)


def _sc_deg(ei2, zeros16):
    @functools.partial(
        pl.kernel,
        out_type=jax.ShapeDtypeStruct((NC, N_PAD, 16), jnp.float32),
        mesh=_mesh,
        scratch_types=[
            pltpu.VMEM((2 * NCHUNK_W, CHUNK), jnp.int32),
            pltpu.VMEM((CHUNK, 16), jnp.float32),
            pltpu.VMEM_SHARED((N_PAD, 16), jnp.float32),
        ],
        compiler_params=_sc_params,
    )
    def k(ei_hbm, z_hbm, out_hbm, idx_v, ones_v, acc):
        cid = lax.axis_index("c")
        sid = lax.axis_index("s")
        wid = sid * NC + cid
        pltpu.sync_copy(z_hbm, acc.at[pl.ds(sid * SLICE, SLICE)])

        @pl.loop(0, CHUNK)
        def _(i):
            ones_v[i] = jnp.full((16,), 1.0, jnp.float32)

        pltpu.sync_copy(
            ei_hbm.at[pl.ds(wid * 2 * NCHUNK_W, 2 * NCHUNK_W)], idx_v)
        plsc.subcore_barrier()

        @pl.loop(0, NCHUNK_W)
        def _(j):
            pltpu.sync_copy(ones_v, acc.at[idx_v.at[2 * j + 1]], add=True)

        plsc.subcore_barrier()
        pltpu.sync_copy(acc.at[pl.ds(sid * SLICE, SLICE)],
                        out_hbm.at[cid].at[pl.ds(sid * SLICE, SLICE)])

    return k(ei2, zeros16)


def _sc_agg(g, ei2, zeros64):
    @functools.partial(
        pl.kernel,
        out_type=jax.ShapeDtypeStruct((NC, N_PAD, HID), jnp.float32),
        mesh=_mesh,
        scratch_types=[
            pltpu.VMEM((2 * NCHUNK_A, CHUNK), jnp.int32),
            [pltpu.VMEM((CHUNK, HID), jnp.bfloat16) for _ in range(NSLOT)],
            [pltpu.VMEM((CHUNK, HID), jnp.float32) for _ in range(NSLOT)],
            pltpu.VMEM_SHARED((N_PAD, HID), jnp.float32),
            [pltpu.SemaphoreType.DMA for _ in range(NSLOT)],
            [pltpu.SemaphoreType.DMA for _ in range(NSLOT)],
        ],
        compiler_params=_sc_params_nl,
    )
    def k(g_hbm, ei_hbm, z_hbm, out_hbm, evi, bbufs, fbufs, acc, gsems, ssems):
        cid = lax.axis_index("c")
        sid = lax.axis_index("s")
        pltpu.sync_copy(z_hbm, acc.at[pl.ds(sid * SLICE, SLICE)])
        plsc.subcore_barrier()

        mask = jnp.full((16,), -65536, jnp.int32)

        def widen(bbuf, fbuf):
            @pl.loop(0, CHUNK)
            def _(r):
                for h in range(2):
                    w = plsc.bitcast(bbuf[r, pl.ds(h * 32, 32)], jnp.int32)
                    fbuf[r, pl.ds(h * 32, 16)] = plsc.bitcast(
                        lax.shift_left(w, 16), jnp.float32)
                    fbuf[r, pl.ds(h * 32 + 16, 16)] = plsc.bitcast(
                        jnp.bitwise_and(w, mask), jnp.float32)

        def run(base, nch):
            pltpu.sync_copy(ei_hbm.at[pl.ds(2 * base, 2 * nch)],
                            evi.at[pl.ds(0, 2 * nch)])
            for b in range(NSLOT):
                pltpu.async_copy(g_hbm.at[evi.at[2 * b]], bbufs[b], gsems[b])

            @pl.loop(0, nch, step=NSLOT)
            def _(j):
                for b in range(NSLOT):
                    jj = j + b
                    pltpu.make_async_copy(
                        g_hbm.at[evi.at[2 * jj]], bbufs[b], gsems[b]).wait()

                    @pl.when(jj >= NSLOT)
                    def _():
                        pltpu.make_async_copy(
                            fbufs[b], acc.at[evi.at[2 * (jj - NSLOT) + 1]],
                            ssems[b]).wait()

                    widen(bbufs[b], fbufs[b])

                    @pl.when(jj + NSLOT < nch)
                    def _():
                        pltpu.async_copy(
                            g_hbm.at[evi.at[2 * (jj + NSLOT)]],
                            bbufs[b], gsems[b])

                    pltpu.async_copy(
                        fbufs[b], acc.at[evi.at[2 * jj + 1]], ssems[b],
                        add=True)

            for b in range(NSLOT):
                pltpu.make_async_copy(
                    fbufs[b], acc.at[evi.at[2 * (nch - NSLOT + b) + 1]],
                    ssems[b]).wait()

        @pl.when(cid == 0)
        def _():
            run(sid * NCHUNK_A, NCHUNK_A)

        @pl.when(cid == 1)
        def _():
            run(C0_CHUNKS + sid * NCHUNK_B, NCHUNK_B)

        plsc.subcore_barrier()
        pltpu.sync_copy(acc.at[pl.ds(sid * SLICE, SLICE)],
                        out_hbm.at[cid].at[pl.ds(sid * SLICE, SLICE)])

    return k(g, ei2, zeros64)


BLK = 1000
GRID = N_NODES // BLK


def _dinv_of(deg):
    safe = jnp.where(deg > 0, deg, 1.0)
    return jnp.where(deg > 0, lax.rsqrt(safe), 0.0)


def _dinv_block(d_ref):
    return _dinv_of(d_ref[0, :, 0] + d_ref[1, :, 0])


def _tc_mm1(x, W1p):
    def body(x_ref, w_ref, o_ref):
        o_ref[...] = jnp.dot(x_ref[...], w_ref[...],
                             preferred_element_type=jnp.float32)

    return pl.pallas_call(
        body,
        grid=(GRID,),
        in_specs=[
            pl.BlockSpec((BLK, D_FEAT), lambda i: (i, 0)),
            pl.BlockSpec((D_FEAT, HID), lambda i: (0, 0)),
        ],
        out_specs=pl.BlockSpec((BLK, HID), lambda i: (i, 0)),
        out_shape=jax.ShapeDtypeStruct((N_NODES, HID), jnp.float32),
    )(x, W1p)


def _tc_scale(h, degp):
    def body(h_ref, d_ref, o_ref):
        dinv = _dinv_block(d_ref)
        o_ref[...] = (h_ref[...] * dinv[:, None]).astype(jnp.bfloat16)

    return pl.pallas_call(
        body,
        grid=(GRID,),
        in_specs=[
            pl.BlockSpec((BLK, HID), lambda i: (i, 0)),
            pl.BlockSpec((NC, BLK, 16), lambda i: (0, i, 0)),
        ],
        out_specs=pl.BlockSpec((BLK, HID), lambda i: (i, 0)),
        out_shape=jax.ShapeDtypeStruct((N_NODES, HID), jnp.bfloat16),
    )(h, degp)


def _tc_mid(part1, degp, b1p):
    def body(p_ref, d_ref, b_ref, o_ref):
        dinv = _dinv_block(d_ref)
        s = p_ref[0] + p_ref[1]
        h = jnp.maximum(s * dinv[:, None] + b_ref[...], 0.0)
        o_ref[...] = (h * dinv[:, None]).astype(jnp.bfloat16)

    return pl.pallas_call(
        body,
        grid=(GRID,),
        in_specs=[
            pl.BlockSpec((NC, BLK, HID), lambda i: (0, i, 0)),
            pl.BlockSpec((NC, BLK, 16), lambda i: (0, i, 0)),
            pl.BlockSpec((1, HID), lambda i: (0, 0)),
        ],
        out_specs=pl.BlockSpec((BLK, HID), lambda i: (i, 0)),
        out_shape=jax.ShapeDtypeStruct((N_NODES, HID), jnp.bfloat16),
    )(part1, degp, b1p)


def _tc_final(part2, degp, W2t, b2c):
    def body(p_ref, d_ref, w_ref, b_ref, o_ref):
        deg = d_ref[0, :, 0] + d_ref[1, :, 0]
        dinv = _dinv_of(deg)
        s = (p_ref[0] + p_ref[1]) * dinv[:, None]
        st = s[:N_NODES]
        o_ref[...] = lax.dot_general(
            w_ref[...], st, (((1,), (1,)), ((), ())),
            preferred_element_type=jnp.float32) + b_ref[...]

    return pl.pallas_call(
        body,
        in_specs=[
            pl.BlockSpec((NC, N_PAD, HID), lambda: (0, 0, 0)),
            pl.BlockSpec((NC, N_PAD, 16), lambda: (0, 0, 0)),
            pl.BlockSpec((HID, HID), lambda: (0, 0)),
            pl.BlockSpec((HID, 1), lambda: (0, 0)),
        ],
        out_specs=pl.BlockSpec((HID, N_NODES), lambda: (0, 0)),
        out_shape=jax.ShapeDtypeStruct((HID, N_NODES), jnp.float32),
    )(part2, degp, W2t, b2c)


def kernel(x, edge_index, W1, b1, W2, b2):
    ei = edge_index.astype(jnp.int32).reshape(2, NCHUNK - 60, CHUNK)
    ei = ei.transpose(1, 0, 2).reshape(2 * (NCHUNK - 60), CHUNK)
    pad = jnp.tile(
        jnp.array([[PAD_ROW], [PAD_COL]], jnp.int32), (60, CHUNK))
    ei2 = jnp.concatenate([ei, pad])

    zeros16 = jnp.zeros((SLICE, 16), jnp.float32)
    zeros64 = jnp.zeros((SLICE, HID), jnp.float32)

    W1p = W1[:, _IPI[_IPI]]
    b1p = b1[_IPI]

    degp = _sc_deg(ei2, zeros16)
    h1x = _tc_mm1(x, W1p)
    g1 = _tc_scale(h1x, degp)
    part1 = _sc_agg(g1, ei2, zeros64)
    g2 = _tc_mid(part1, degp, b1p.reshape(1, HID))
    part2 = _sc_agg(g2, ei2, zeros64)
    outT = _tc_final(part2, degp, W2.T, b2.reshape(HID, 1))
    return outT.T


# speedup vs baseline: 1.0212x; 1.0212x over previous
import dataclasses
import functools

import jax
import jax.numpy as jnp
import numpy as np
from jax import lax
from jax.experimental import pallas as pl
from jax.experimental.pallas import tpu as pltpu
from jax.experimental.pallas import tpu_sc as plsc

N_NODES = 10000
N_EDGES = 320000
D_FEAT = 128
HID = 64

NC = 2
NS = 16
NW = NC * NS
CHUNK = 128
NCHUNK = 2560
NCHUNK_W = NCHUNK // NW
NCHUNK_A = 96
NCHUNK_B = 64
C0_CHUNKS = NS * NCHUNK_A
PAD_ROW = 0
PAD_COL = N_NODES
N_PAD = 10240
SLICE = N_PAD // NS
NSLOT = 4

_pi32 = np.concatenate([np.arange(0, 32, 2), np.arange(1, 32, 2)])
_PI = np.concatenate([_pi32, 32 + _pi32])
_IPI = np.argsort(_PI)

_mesh = plsc.VectorSubcoreMesh(core_axis_name="c", subcore_axis_name="s")
_sc_params = pltpu.CompilerParams(use_tc_tiling_on_sc=False)
_sc_params_nl = dataclasses.replace(_sc_params, needs_layout_passes=False)


def _sc_deg(ei2, zeros16):
    @functools.partial(
        pl.kernel,
        out_type=jax.ShapeDtypeStruct((NC, N_PAD, 16), jnp.float32),
        mesh=_mesh,
        scratch_types=[
            pltpu.VMEM((2 * NCHUNK_W, CHUNK), jnp.int32),
            pltpu.VMEM((CHUNK, 16), jnp.float32),
            pltpu.VMEM_SHARED((N_PAD, 16), jnp.float32),
        ],
        compiler_params=_sc_params,
    )
    def k(ei_hbm, z_hbm, out_hbm, idx_v, ones_v, acc):
        cid = lax.axis_index("c")
        sid = lax.axis_index("s")
        wid = sid * NC + cid
        pltpu.sync_copy(z_hbm, acc.at[pl.ds(sid * SLICE, SLICE)])

        @pl.loop(0, CHUNK)
        def _(i):
            ones_v[i] = jnp.full((16,), 1.0, jnp.float32)

        pltpu.sync_copy(
            ei_hbm.at[pl.ds(wid * 2 * NCHUNK_W, 2 * NCHUNK_W)], idx_v)
        plsc.subcore_barrier()

        @pl.loop(0, NCHUNK_W)
        def _(j):
            pltpu.sync_copy(ones_v, acc.at[idx_v.at[2 * j + 1]], add=True)

        plsc.subcore_barrier()
        pltpu.sync_copy(acc.at[pl.ds(sid * SLICE, SLICE)],
                        out_hbm.at[cid].at[pl.ds(sid * SLICE, SLICE)])

    return k(ei2, zeros16)


def _sc_agg(g, ei2, zeros64):
    @functools.partial(
        pl.kernel,
        out_type=jax.ShapeDtypeStruct((NC, N_PAD, HID), jnp.float32),
        mesh=_mesh,
        scratch_types=[
            pltpu.VMEM((2 * NCHUNK_A, CHUNK), jnp.int32),
            [pltpu.VMEM((CHUNK, HID), jnp.bfloat16) for _ in range(NSLOT)],
            [pltpu.VMEM((CHUNK, HID), jnp.float32) for _ in range(NSLOT)],
            pltpu.VMEM_SHARED((N_PAD, HID), jnp.float32),
            [pltpu.SemaphoreType.DMA for _ in range(NSLOT)],
            [pltpu.SemaphoreType.DMA for _ in range(NSLOT)],
        ],
        compiler_params=_sc_params_nl,
    )
    def k(g_hbm, ei_hbm, z_hbm, out_hbm, evi, bbufs, fbufs, acc, gsems, ssems):
        cid = lax.axis_index("c")
        sid = lax.axis_index("s")
        pltpu.sync_copy(z_hbm, acc.at[pl.ds(sid * SLICE, SLICE)])
        plsc.subcore_barrier()

        mask = jnp.full((16,), -65536, jnp.int32)

        def widen(bbuf, fbuf):
            @pl.loop(0, CHUNK)
            def _(r):
                for h in range(2):
                    w = plsc.bitcast(bbuf[r, pl.ds(h * 32, 32)], jnp.int32)
                    fbuf[r, pl.ds(h * 32, 16)] = plsc.bitcast(
                        lax.shift_left(w, 16), jnp.float32)
                    fbuf[r, pl.ds(h * 32 + 16, 16)] = plsc.bitcast(
                        jnp.bitwise_and(w, mask), jnp.float32)

        def run(base, nch):
            pltpu.sync_copy(ei_hbm.at[pl.ds(2 * base, 2 * nch)],
                            evi.at[pl.ds(0, 2 * nch)])
            for b in range(NSLOT):
                pltpu.async_copy(g_hbm.at[evi.at[2 * b]], bbufs[b], gsems[b])

            @pl.loop(0, nch, step=NSLOT)
            def _(j):
                for b in range(NSLOT):
                    jj = j + b
                    pltpu.make_async_copy(
                        g_hbm.at[evi.at[2 * jj]], bbufs[b], gsems[b]).wait()

                    @pl.when(jj >= NSLOT)
                    def _():
                        pltpu.make_async_copy(
                            fbufs[b], acc.at[evi.at[2 * (jj - NSLOT) + 1]],
                            ssems[b]).wait()

                    widen(bbufs[b], fbufs[b])

                    @pl.when(jj + NSLOT < nch)
                    def _():
                        pltpu.async_copy(
                            g_hbm.at[evi.at[2 * (jj + NSLOT)]],
                            bbufs[b], gsems[b])

                    pltpu.async_copy(
                        fbufs[b], acc.at[evi.at[2 * jj + 1]], ssems[b],
                        add=True)

            for b in range(NSLOT):
                pltpu.make_async_copy(
                    fbufs[b], acc.at[evi.at[2 * (nch - NSLOT + b) + 1]],
                    ssems[b]).wait()

        @pl.when(cid == 0)
        def _():
            run(sid * NCHUNK_A, NCHUNK_A)

        @pl.when(cid == 1)
        def _():
            run(C0_CHUNKS + sid * NCHUNK_B, NCHUNK_B)

        plsc.subcore_barrier()
        pltpu.sync_copy(acc.at[pl.ds(sid * SLICE, SLICE)],
                        out_hbm.at[cid].at[pl.ds(sid * SLICE, SLICE)])

    return k(g, ei2, zeros64)


BLK = 1000
GRID = N_NODES // BLK


def _dinv_of(deg):
    safe = jnp.where(deg > 0, deg, 1.0)
    return jnp.where(deg > 0, lax.rsqrt(safe), 0.0)


def _dinv_block(d_ref):
    return _dinv_of(d_ref[0, :, 0] + d_ref[1, :, 0])


def _tc_mm1(x, W1p):
    def body(x_ref, w_ref, o_ref):
        o_ref[...] = jnp.dot(x_ref[...], w_ref[...],
                             preferred_element_type=jnp.float32)

    return pl.pallas_call(
        body,
        grid=(GRID,),
        in_specs=[
            pl.BlockSpec((BLK, D_FEAT), lambda i: (i, 0)),
            pl.BlockSpec((D_FEAT, HID), lambda i: (0, 0)),
        ],
        out_specs=pl.BlockSpec((BLK, HID), lambda i: (i, 0)),
        out_shape=jax.ShapeDtypeStruct((N_NODES, HID), jnp.float32),
    )(x, W1p)


def _tc_scale(h, degp):
    def body(h_ref, d_ref, o_ref):
        dinv = _dinv_block(d_ref)
        o_ref[...] = (h_ref[...] * dinv[:, None]).astype(jnp.bfloat16)

    return pl.pallas_call(
        body,
        grid=(GRID,),
        in_specs=[
            pl.BlockSpec((BLK, HID), lambda i: (i, 0)),
            pl.BlockSpec((NC, BLK, 16), lambda i: (0, i, 0)),
        ],
        out_specs=pl.BlockSpec((BLK, HID), lambda i: (i, 0)),
        out_shape=jax.ShapeDtypeStruct((N_NODES, HID), jnp.bfloat16),
    )(h, degp)


def _tc_mid(part1, degp, b1p):
    def body(p_ref, d_ref, b_ref, o_ref):
        dinv = _dinv_block(d_ref)
        s = p_ref[0] + p_ref[1]
        h = jnp.maximum(s * dinv[:, None] + b_ref[...], 0.0)
        o_ref[...] = (h * dinv[:, None]).astype(jnp.bfloat16)

    return pl.pallas_call(
        body,
        grid=(GRID,),
        in_specs=[
            pl.BlockSpec((NC, BLK, HID), lambda i: (0, i, 0)),
            pl.BlockSpec((NC, BLK, 16), lambda i: (0, i, 0)),
            pl.BlockSpec((1, HID), lambda i: (0, 0)),
        ],
        out_specs=pl.BlockSpec((BLK, HID), lambda i: (i, 0)),
        out_shape=jax.ShapeDtypeStruct((N_NODES, HID), jnp.bfloat16),
    )(part1, degp, b1p)


def _tc_final(part2, degp, W2t, b2c):
    def body(p_ref, d_ref, w_ref, b_ref, o_ref):
        deg = d_ref[0, :, 0] + d_ref[1, :, 0]
        dinv = _dinv_of(deg)
        s = (p_ref[0] + p_ref[1]) * dinv[:, None]
        st = s[:N_NODES]
        o_ref[...] = lax.dot_general(
            w_ref[...], st, (((1,), (1,)), ((), ())),
            preferred_element_type=jnp.float32) + b_ref[...]

    return pl.pallas_call(
        body,
        in_specs=[
            pl.BlockSpec((NC, N_PAD, HID), lambda: (0, 0, 0)),
            pl.BlockSpec((NC, N_PAD, 16), lambda: (0, 0, 0)),
            pl.BlockSpec((HID, HID), lambda: (0, 0)),
            pl.BlockSpec((HID, 1), lambda: (0, 0)),
        ],
        out_specs=pl.BlockSpec((HID, N_NODES), lambda: (0, 0)),
        out_shape=jax.ShapeDtypeStruct((HID, N_NODES), jnp.float32),
    )(part2, degp, W2t, b2c)


def kernel(x, edge_index, W1, b1, W2, b2):
    ei = edge_index.astype(jnp.int32).reshape(2, NCHUNK - 60, CHUNK)
    ei = ei.transpose(1, 0, 2).reshape(2 * (NCHUNK - 60), CHUNK)
    pad = jnp.tile(
        jnp.array([[PAD_ROW], [PAD_COL]], jnp.int32), (60, CHUNK))
    ei2 = jnp.concatenate([ei, pad])

    zeros16 = jnp.zeros((SLICE, 16), jnp.float32)
    zeros64 = jnp.zeros((SLICE, HID), jnp.float32)

    W1p = W1[:, _IPI[_IPI]]
    b1p = b1[_IPI]

    degp = _sc_deg(ei2, zeros16)
    h1x = _tc_mm1(x, W1p)
    g1 = _tc_scale(h1x, degp)
    part1 = _sc_agg(g1, ei2, zeros64)
    g2 = _tc_mid(part1, degp, b1p.reshape(1, HID))
    part2 = _sc_agg(g2, ei2, zeros64)
    outT = _tc_final(part2, degp, W2.T, b2.reshape(HID, 1))
    return outT.T
